# baseline (speedup 1.0000x reference)
import jax
import jax.numpy as jnp
from jax.experimental import pallas as pl
from jax.experimental.pallas import tpu as pltpu


def _matmul_kernel(feat_ref, weight_ref, out_ref):
    out_ref[...] = jax.lax.dot_general(
        feat_ref[...], weight_ref[...],
        dimension_numbers=(((1,), (1,)), ((), ())),
        preferred_element_type=jnp.float32,
    )


def kernel(feat, weight):
    batch, _ = feat.shape
    num_classes, _ = weight.shape
    return pl.pallas_call(
        _matmul_kernel,
        out_shape=jax.ShapeDtypeStruct((batch, num_classes), jnp.float32),
    )(feat, weight)
